# baseline (speedup 1.0000x reference)
import functools

import jax
import jax.numpy as jnp
from jax.experimental import pallas as pl
from jax.experimental.pallas import tpu as pltpu
from jax.experimental.pallas import tpu_sc as plsc

_BIG = 3.0e38


def _dotd(a, b):
    return jax.lax.dot_general(a.astype(jnp.bfloat16), b.astype(jnp.bfloat16),
                               (((1,), (0,)), ((), ())),
                               preferred_element_type=jnp.float32)


def _bsplit(b):
    bh = b.astype(jnp.bfloat16)
    r = b - bh.astype(jnp.float32)
    bm = r.astype(jnp.bfloat16)
    bl = (r - bm.astype(jnp.float32)).astype(jnp.bfloat16)
    return jnp.concatenate([bh, bm, bl], axis=1), b.shape[1]


def _dot2(a, parts):
    bcat, w = parts
    t3 = jax.lax.dot_general(a.astype(jnp.bfloat16), bcat,
                             (((1,), (0,)), ((), ())),
                             preferred_element_type=jnp.float32)
    return (t3[:, 0:w] + t3[:, w:2 * w]) + t3[:, 2 * w:3 * w]


def _dist_stacked(qp, pct, nb, q_blk, row_base):
    qt = qp.shape[0]
    q2 = jnp.sum(qp * qp, axis=1, keepdims=True)
    if nb == 1:
        p2 = jnp.sum(pct * pct, axis=0, keepdims=True)
        return q2 + p2 - 2.0 * _dotd(qp, pct), None
    riota = jax.lax.broadcasted_iota(jnp.int32, (qt, 1), 0) + row_base
    bq = riota // q_blk
    aq = jnp.concatenate(
        [jnp.where(bq == b2, qp, 0.0) for b2 in range(nb)], axis=1)
    cross = _dotd(aq, pct)
    p2r = jnp.sum(pct[0:3] * pct[0:3], axis=0, keepdims=True)
    for b2 in range(1, nb):
        pb = jnp.sum(pct[3 * b2:3 * b2 + 3] * pct[3 * b2:3 * b2 + 3],
                     axis=0, keepdims=True)
        p2r = jnp.where(bq == b2, pb, p2r)
    return q2 + p2r - 2.0 * cross, bq


def _argmin(dist, iota):
    m = jnp.min(dist, axis=1, keepdims=True)
    cand = jnp.where(dist <= m, iota, jnp.int32(0x7FFFFFFF))
    am = jnp.min(cand, axis=1, keepdims=True)
    return m, am


def _conv_kernel(s_ref, pct_ref, qp_ref, *wrefs,
                 n_layers, k_nn, d_feat, fold, nb, q_blk):
    wrefs, out_ref = wrefs[:-1], wrefs[-1]
    layers = [(wrefs[3 * i][...], wrefs[3 * i + 1][...], wrefs[3 * i + 2][...])
              for i in range(n_layers)]
    s = s_ref[0]
    pct = pct_ref[0]
    qp = qp_ref[0]
    n = pct.shape[1]
    qt = qp.shape[0]
    d = d_feat
    d_out = layers[-1][0].shape[1]

    dist, bq = _dist_stacked(qp, pct, nb, q_blk, pl.program_id(1) * qt)
    iota = jax.lax.broadcasted_iota(jnp.int32, (qt, n), 1)
    if fold:
        iota_c = jax.lax.broadcasted_iota(jnp.int32, (qt, nb * n // 8), 1)
    else:
        iota_g = jax.lax.broadcasted_iota(jnp.int32, (qt, nb * n), 1)
    s_parts = _bsplit(s)
    w1 = layers[0][0]

    def body(_, carry):
        dist, acc = carry
        _, am = _argmin(dist, iota)
        am_g = am if nb == 1 else am + bq * n
        if fold:
            rowhot = (iota_c == jax.lax.shift_right_logical(am_g, 3))
            t = _dot2(rowhot.astype(jnp.float32), s_parts)
            sub = jnp.bitwise_and(am_g, 7)
            g = t[:, 0:d]
            for j in range(1, 8):
                g = jnp.where(sub == j, t[:, j * d:(j + 1) * d], g)
        else:
            g = _dot2((iota_g == am_g).astype(jnp.float32), s_parts)
        g_cat = jnp.concatenate([g[:, :3] - qp, g[:, 3:]], axis=1)
        h = jnp.maximum(_dotd(g_cat, w1) * layers[0][1] + layers[0][2], 0.0)
        for w, ga, be in layers[1:]:
            h = jnp.maximum(_dotd(h, w) * ga + be, 0.0)
        acc = jnp.maximum(acc, h)
        dist = jnp.where(iota == am, _BIG, dist)
        return dist, acc

    _, acc = jax.lax.fori_loop(
        0, k_nn, body, (dist, jnp.zeros((qt, d_out), jnp.float32)),
        unroll=4)
    out_ref[0] = acc


def _conv_stage(s, pct, qp, layers, k_nn, q_tile, stack=False):
    b, n, d = s.shape
    q = qp.shape[1]
    d_out = layers[-1][0].shape[1]
    nb = b if stack else 1
    if stack:
        s = s.reshape(1, b * n, d)
        pct = pct.reshape(1, 3 * b, n)
        qp = qp.reshape(1, b * q, 3)
    bq_tot = qp.shape[1]
    q_tile = min(q_tile, bq_tot)
    fold = nb * n >= 1024
    if fold:
        s = s.reshape(s.shape[0], nb * n // 8, 8 * d)
    wargs = []
    for (w, ga, be) in layers:
        wargs += [w, ga.reshape(1, -1), be.reshape(1, -1)]
    grid = (s.shape[0], bq_tot // q_tile)
    full = lambda arr: pl.BlockSpec(arr.shape, lambda bi, qi: (0,) * arr.ndim)
    out = pl.pallas_call(
        functools.partial(_conv_kernel, n_layers=len(layers), k_nn=k_nn,
                          d_feat=d, fold=fold, nb=nb, q_blk=q),
        grid=grid,
        in_specs=[
            pl.BlockSpec((1,) + s.shape[1:], lambda bi, qi: (bi, 0, 0)),
            pl.BlockSpec((1,) + pct.shape[1:], lambda bi, qi: (bi, 0, 0)),
            pl.BlockSpec((1, q_tile, 3), lambda bi, qi: (bi, qi, 0)),
        ] + [full(a) for a in wargs],
        out_specs=pl.BlockSpec((1, q_tile, d_out), lambda bi, qi: (bi, qi, 0)),
        out_shape=jax.ShapeDtypeStruct((s.shape[0], bq_tot, d_out), jnp.float32),
    )(s, pct, qp, *wargs)
    return out.reshape(b, q, d_out)


def _knn_kernel(pct_ref, qp_ref, idx_ref, *, k_nn):
    pct = pct_ref[0]
    qp = qp_ref[0]
    n = pct.shape[1]
    qt = qp.shape[0]
    dist, _ = _dist_stacked(qp, pct, 1, 0, 0)
    iota = jax.lax.broadcasted_iota(jnp.int32, (qt, n), 1)
    kio = jax.lax.broadcasted_iota(jnp.int32, (qt, k_nn), 1)
    off = pl.program_id(0) * n

    def body(k, carry):
        dist, idxs = carry
        _, am = _argmin(dist, iota)
        idxs = jnp.where(kio == k, am + off, idxs)
        dist = jnp.where(iota == am, _BIG, dist)
        return dist, idxs

    _, idxs = jax.lax.fori_loop(
        0, k_nn, body, (dist, jnp.zeros((qt, k_nn), jnp.int32)), unroll=4)
    idx_ref[0] = idxs


def _knn_stage(pct, qp, k_nn, q_tile):
    b = pct.shape[0]
    n = pct.shape[2]
    q = qp.shape[1]
    q_tile = min(q_tile, q)
    return pl.pallas_call(
        functools.partial(_knn_kernel, k_nn=k_nn),
        grid=(b, q // q_tile),
        in_specs=[pl.BlockSpec((1, 3, n), lambda bi, qi: (bi, 0, 0)),
                  pl.BlockSpec((1, q_tile, 3), lambda bi, qi: (bi, qi, 0))],
        out_specs=pl.BlockSpec((1, q_tile, k_nn), lambda bi, qi: (bi, qi, 0)),
        out_shape=jax.ShapeDtypeStruct((b, q, k_nn), jnp.int32),
    )(pct, qp)


def _sc_gather(table, idx_flat, window=128):
    m = idx_flat.shape[0]
    d = table.shape[1]
    mesh = plsc.VectorSubcoreMesh(core_axis_name='core',
                                  subcore_axis_name='subcore',
                                  num_cores=2, num_subcores=16)
    idx2 = idx_flat.reshape(1, m)

    @functools.partial(
        pl.kernel,
        out_type=jax.ShapeDtypeStruct((m, d), table.dtype),
        mesh=mesh)
    def gk(x_hbm, i_hbm, o_hbm):
        def body(i_vmem, o_vmem):
            pltpu.sync_copy(x_hbm.at[i_vmem.at[0]], o_vmem)

        pltpu.emit_pipeline(
            body,
            grid=(m // window,),
            in_specs=[pl.BlockSpec((1, window), index_map=lambda i: (0, i))],
            out_specs=[pl.BlockSpec((window, d), index_map=lambda i: (i, 0))],
            core_axis_name='subcore',
            dimension_semantics=(pltpu.PARALLEL,),
        )(i_hbm, o_hbm)

    return gk(table, idx2)


def _mlpmax_kernel(g_ref, qp_ref, *wrefs, n_layers, k_nn, d_feat):
    wrefs, out_ref = wrefs[:-1], wrefs[-1]
    layers = [(wrefs[3 * i][...], wrefs[3 * i + 1][...], wrefs[3 * i + 2][...])
              for i in range(n_layers)]
    g = g_ref[0]
    qp = qp_ref[0]
    qt = qp.shape[0]
    g3 = g.reshape(qt, k_nn, g.shape[1])
    gc = jnp.concatenate([g3[:, :, :3] - qp[:, None, :],
                          g3[:, :, 3:d_feat]], axis=2)
    h = gc.reshape(qt * k_nn, d_feat)
    for w, ga, be in layers:
        h = jnp.maximum(_dotd(h, w) * ga + be, 0.0)
    d_out = layers[-1][0].shape[1]
    out_ref[0] = jnp.max(h.reshape(qt, k_nn, d_out), axis=1)


def _conv_stage_sc(s, pct, qp, layers, k_nn, q_tile, mlp_tile):
    b, n, d = s.shape
    q = qp.shape[1]
    d_out = layers[-1][0].shape[1]
    idx = _knn_stage(pct, qp, k_nn, q_tile)
    d_pad = 128
    s_pad = jnp.concatenate(
        [s, jnp.zeros((b, n, d_pad - d), jnp.float32)], axis=2)
    g = _sc_gather(s_pad.reshape(b * n, d_pad), idx.reshape(b * q * k_nn))
    g = g.reshape(b, q * k_nn, d_pad)
    wargs = []
    for (w, ga, be) in layers:
        wargs += [w, ga.reshape(1, -1), be.reshape(1, -1)]
    full = lambda arr: pl.BlockSpec(arr.shape, lambda bi, qi: (0,) * arr.ndim)
    return pl.pallas_call(
        functools.partial(_mlpmax_kernel, n_layers=len(layers), k_nn=k_nn,
                          d_feat=d),
        grid=(b, q // mlp_tile),
        in_specs=[
            pl.BlockSpec((1, mlp_tile * k_nn, d_pad),
                         lambda bi, qi: (bi, qi, 0)),
            pl.BlockSpec((1, mlp_tile, 3), lambda bi, qi: (bi, qi, 0)),
        ] + [full(a) for a in wargs],
        out_specs=pl.BlockSpec((1, mlp_tile, d_out), lambda bi, qi: (bi, qi, 0)),
        out_shape=jax.ShapeDtypeStruct((b, q, d_out), jnp.float32),
    )(g, qp, *wargs)


def _knn3_kernel(pct_ref, qp_ref, idx_ref, m_ref, *, k_nn):
    pct = pct_ref[0]
    qp = qp_ref[0]
    n = pct.shape[1]
    qt = qp.shape[0]
    dist, _ = _dist_stacked(qp, pct, 1, 0, 0)
    iota = jax.lax.broadcasted_iota(jnp.int32, (qt, n), 1)
    kio = jax.lax.broadcasted_iota(jnp.int32, (qt, k_nn), 1)
    off = pl.program_id(0) * n
    idxs = jnp.zeros((qt, k_nn), jnp.int32)
    ms = jnp.zeros((qt, k_nn), jnp.float32)
    for k in range(k_nn):
        m, am = _argmin(dist, iota)
        idxs = jnp.where(kio == k, am + off, idxs)
        ms = jnp.where(kio == k, m, ms)
        dist = jnp.where(iota == am, _BIG, dist)
    idx_ref[0] = idxs
    m_ref[0] = ms


def _knn3_stage(pct, qp, k_nn, q_tile):
    b = pct.shape[0]
    n = pct.shape[2]
    q = qp.shape[1]
    q_tile = min(q_tile, q)
    return pl.pallas_call(
        functools.partial(_knn3_kernel, k_nn=k_nn),
        grid=(b, q // q_tile),
        in_specs=[pl.BlockSpec((1, 3, n), lambda bi, qi: (bi, 0, 0)),
                  pl.BlockSpec((1, q_tile, 3), lambda bi, qi: (bi, qi, 0))],
        out_specs=[
            pl.BlockSpec((1, q_tile, k_nn), lambda bi, qi: (bi, qi, 0)),
            pl.BlockSpec((1, q_tile, k_nn), lambda bi, qi: (bi, qi, 0))],
        out_shape=[jax.ShapeDtypeStruct((b, q, k_nn), jnp.int32),
                   jax.ShapeDtypeStruct((b, q, k_nn), jnp.float32)],
    )(pct, qp)


def _dinterp_kernel(f_ref, m_ref, *wrefs, n_layers, k_nn, has_ff):
    wrefs, out_ref = wrefs[:-1], wrefs[-1]
    off = 0
    if has_ff:
        ff = wrefs[0][0]
        off = 1
    layers = [(wrefs[off + 3 * i][...], wrefs[off + 3 * i + 1][...],
               wrefs[off + 3 * i + 2][...]) for i in range(n_layers)]
    f = f_ref[0]
    ms = m_ref[0]
    cc = f.shape[1]
    qt = ms.shape[0]
    f3 = f.reshape(qt, k_nn, cc)
    w = 1.0 / jnp.maximum(ms, 1e-10)
    num = jnp.sum(f3 * w[:, :, None], axis=1)
    den = jnp.sum(w, axis=1, keepdims=True)
    interp = num / den
    w1, g1, b1 = layers[0]
    pre = _dotd(interp, w1[:cc, :])
    if has_ff:
        pre = pre + _dotd(ff, w1[cc:, :])
    h = jnp.maximum(pre * g1 + b1, 0.0)
    for w_, ga, be in layers[1:]:
        h = jnp.maximum(_dotd(h, w_) * ga + be, 0.0)
    out_ref[0] = h


def _deconv_stage_sc(fc, pct, qp, ff, layers, q_tile, mlp_tile):
    b, nc, cc = fc.shape
    q = qp.shape[1]
    d_out = layers[-1][0].shape[1]
    k_nn = 3
    idx, ms = _knn3_stage(pct, qp, k_nn, q_tile)
    g = _sc_gather(fc.reshape(b * nc, cc), idx.reshape(b * q * k_nn))
    g = g.reshape(b, q * k_nn, cc)
    wargs = []
    for (w, ga, be) in layers:
        wargs += [w, ga.reshape(1, -1), be.reshape(1, -1)]
    has_ff = ff is not None
    full = lambda arr: pl.BlockSpec(arr.shape, lambda bi, qi: (0,) * arr.ndim)
    in_specs = [
        pl.BlockSpec((1, mlp_tile * k_nn, cc), lambda bi, qi: (bi, qi, 0)),
        pl.BlockSpec((1, mlp_tile, k_nn), lambda bi, qi: (bi, qi, 0)),
    ]
    args = [g, ms]
    if has_ff:
        in_specs.append(pl.BlockSpec((1, mlp_tile, ff.shape[2]),
                                     lambda bi, qi: (bi, qi, 0)))
        args.append(ff)
    in_specs += [full(a) for a in wargs]
    args += wargs
    return pl.pallas_call(
        functools.partial(_dinterp_kernel, n_layers=len(layers), k_nn=k_nn,
                          has_ff=has_ff),
        grid=(b, q // mlp_tile),
        in_specs=in_specs,
        out_specs=pl.BlockSpec((1, mlp_tile, d_out), lambda bi, qi: (bi, qi, 0)),
        out_shape=jax.ShapeDtypeStruct((b, q, d_out), jnp.float32),
    )(*args)


def _deconv_kernel(fc_ref, pct_ref, qp_ref, *wrefs,
                   n_layers, has_ff, nb, q_blk):
    wrefs, out_ref = wrefs[:-1], wrefs[-1]
    off = 0
    if has_ff:
        ff = wrefs[0][0]
        off = 1
    layers = [(wrefs[off + 3 * i][...], wrefs[off + 3 * i + 1][...],
               wrefs[off + 3 * i + 2][...]) for i in range(n_layers)]
    fc = fc_ref[0]
    pct = pct_ref[0]
    qp = qp_ref[0]
    nc = pct.shape[1]
    cc = fc.shape[1]
    qt = qp.shape[0]

    dist, bq = _dist_stacked(qp, pct, nb, q_blk, pl.program_id(1) * qt)
    iota = jax.lax.broadcasted_iota(jnp.int32, (qt, nc), 1)
    iota_g = jax.lax.broadcasted_iota(jnp.int32, (qt, nb * nc), 1)
    fc_parts = _bsplit(fc)
    num = jnp.zeros((qt, cc), jnp.float32)
    den = jnp.zeros((qt, 1), jnp.float32)
    for _ in range(3):
        m, am = _argmin(dist, iota)
        am_g = am if nb == 1 else am + bq * nc
        f = _dot2((iota_g == am_g).astype(jnp.float32), fc_parts)
        w = 1.0 / jnp.maximum(m, 1e-10)
        num = num + w * f
        den = den + w
        dist = jnp.where(iota == am, _BIG, dist)
    interp = num / den

    w1, g1, b1 = layers[0]
    pre = _dotd(interp, w1[:cc, :])
    if has_ff:
        pre = pre + _dotd(ff, w1[cc:, :])
    h = jnp.maximum(pre * g1 + b1, 0.0)
    for w, ga, be in layers[1:]:
        h = jnp.maximum(_dotd(h, w) * ga + be, 0.0)
    out_ref[0] = h


def _deconv_stage(fc, pct, qp, ff, layers, q_tile, stack=False):
    b, nc, cc = fc.shape
    q = qp.shape[1]
    d_out = layers[-1][0].shape[1]
    nb = b if stack else 1
    if stack:
        fc = fc.reshape(1, b * nc, cc)
        pct = pct.reshape(1, 3 * b, nc)
        qp = qp.reshape(1, b * q, 3)
        if ff is not None:
            ff = ff.reshape(1, b * q, ff.shape[2])
    bq_tot = qp.shape[1]
    q_tile = min(q_tile, bq_tot)
    wargs = []
    for (w, ga, be) in layers:
        wargs += [w, ga.reshape(1, -1), be.reshape(1, -1)]
    has_ff = ff is not None
    grid = (fc.shape[0] if not stack else 1, bq_tot // q_tile)
    full = lambda arr: pl.BlockSpec(arr.shape, lambda bi, qi: (0,) * arr.ndim)
    in_specs = [
        pl.BlockSpec((1,) + fc.shape[1:], lambda bi, qi: (bi, 0, 0)),
        pl.BlockSpec((1,) + pct.shape[1:], lambda bi, qi: (bi, 0, 0)),
        pl.BlockSpec((1, q_tile, 3), lambda bi, qi: (bi, qi, 0)),
    ]
    args = [fc, pct, qp]
    if has_ff:
        in_specs.append(pl.BlockSpec((1, q_tile, ff.shape[2]),
                                     lambda bi, qi: (bi, qi, 0)))
        args.append(ff)
    in_specs += [full(a) for a in wargs]
    args += wargs
    out = pl.pallas_call(
        functools.partial(_deconv_kernel, n_layers=len(layers), has_ff=has_ff,
                          nb=nb, q_blk=q),
        grid=grid,
        in_specs=in_specs,
        out_specs=pl.BlockSpec((1, q_tile, d_out), lambda bi, qi: (bi, qi, 0)),
        out_shape=jax.ShapeDtypeStruct((fc.shape[0], bq_tot, d_out),
                                       jnp.float32),
    )(*args)
    return out.reshape(b, q, d_out)


def _head_kernel(f_ref, *wrefs, n_layers):
    wrefs, out_ref = wrefs[:-1], wrefs[-1]
    layers = [(wrefs[3 * i][...], wrefs[3 * i + 1][...], wrefs[3 * i + 2][...])
              for i in range(n_layers)]
    fcw = wrefs[3 * n_layers][...]
    fcb = wrefs[3 * n_layers + 1][...]
    h = f_ref[0]
    for w, ga, be in layers:
        h = jnp.maximum(_dotd(h, w) * ga + be, 0.0)
    out_ref[0] = _dotd(h, fcw) + fcb


def _head_stage(f, layers, fcw, fcb, q_tile):
    b, q, c = f.shape
    q_tile = min(q_tile, q)
    d_out = fcw.shape[1]
    wargs = []
    for (w, ga, be) in layers:
        wargs += [w, ga.reshape(1, -1), be.reshape(1, -1)]
    wargs += [fcw, fcb.reshape(1, -1)]
    full = lambda arr: pl.BlockSpec(arr.shape, lambda bi, qi: (0,) * arr.ndim)
    return pl.pallas_call(
        functools.partial(_head_kernel, n_layers=len(layers)),
        grid=(b, q // q_tile),
        in_specs=[pl.BlockSpec((1, q_tile, c), lambda bi, qi: (bi, qi, 0))]
                 + [full(a) for a in wargs],
        out_specs=pl.BlockSpec((1, q_tile, d_out), lambda bi, qi: (bi, qi, 0)),
        out_shape=jax.ShapeDtypeStruct((b, q, d_out), jnp.float32),
    )(f, *wargs)


def kernel(x, params):
    xt = jnp.transpose(x, (0, 2, 1))
    p0 = xt[..., :3]
    pct0 = x[:, :3]

    def ch(p):
        return jnp.transpose(p, (0, 2, 1))

    q1 = p0[:, ::4]
    f1 = _conv_stage_sc(xt, pct0, q1, params['conv1'], 32, 512, 256)
    s1 = jnp.concatenate([q1, f1], axis=-1)
    q2 = q1[:, ::4]
    f2 = _conv_stage_sc(s1, ch(q1), q2, params['conv2'], 32, 256, 256)
    s2 = jnp.concatenate([q2, f2], axis=-1)
    q3 = q2[:, ::4]
    f3 = _conv_stage(s2, ch(q2), q3, params['conv3'], 32, 256,
                     stack=True)
    s3 = jnp.concatenate([q3, f3], axis=-1)
    q4 = q3[:, ::4]
    f4 = _conv_stage(s3, ch(q3), q4, params['conv4'], 32, 64,
                     stack=True)

    g3 = _deconv_stage(f4, ch(q4), q3, f3, params['dconv1'], 256,
                       stack=True)
    g2 = _deconv_stage(g3, ch(q3), q2, f2, params['dconv2'], 1024,
                       stack=True)
    g1 = _deconv_stage(g2, ch(q2), q1, f1, params['dconv3'], 4096,
                       stack=True)
    g0 = _deconv_stage_sc(g1, ch(q1), p0, None, params['dconv4'], 1024, 1024)

    out = _head_stage(g0, params['mlp'], params['fc_w'], params['fc_b'], 1024)
    return jnp.transpose(out, (0, 2, 1))


# speedup vs baseline: 1.0519x; 1.0519x over previous
import functools

import jax
import jax.numpy as jnp
from jax.experimental import pallas as pl
from jax.experimental.pallas import tpu as pltpu
from jax.experimental.pallas import tpu_sc as plsc

_BIG = 3.0e38


def _dotd(a, b):
    return jax.lax.dot_general(a.astype(jnp.bfloat16), b.astype(jnp.bfloat16),
                               (((1,), (0,)), ((), ())),
                               preferred_element_type=jnp.float32)


def _bsplit(b):
    bh = b.astype(jnp.bfloat16)
    r = b - bh.astype(jnp.float32)
    bm = r.astype(jnp.bfloat16)
    bl = (r - bm.astype(jnp.float32)).astype(jnp.bfloat16)
    return jnp.concatenate([bh, bm, bl], axis=1), b.shape[1]


def _dot2(a, parts):
    bcat, w = parts
    t3 = jax.lax.dot_general(a.astype(jnp.bfloat16), bcat,
                             (((1,), (0,)), ((), ())),
                             preferred_element_type=jnp.float32)
    return (t3[:, 0:w] + t3[:, w:2 * w]) + t3[:, 2 * w:3 * w]


def _dist_stacked(qp, pct, nb, q_blk, row_base):
    qt = qp.shape[0]
    q2 = jnp.sum(qp * qp, axis=1, keepdims=True)
    if nb == 1:
        p2 = jnp.sum(pct * pct, axis=0, keepdims=True)
        return q2 + p2 - 2.0 * _dotd(qp, pct), None
    riota = jax.lax.broadcasted_iota(jnp.int32, (qt, 1), 0) + row_base
    bq = riota // q_blk
    aq = jnp.concatenate(
        [jnp.where(bq == b2, qp, 0.0) for b2 in range(nb)], axis=1)
    cross = _dotd(aq, pct)
    p2r = jnp.sum(pct[0:3] * pct[0:3], axis=0, keepdims=True)
    for b2 in range(1, nb):
        pb = jnp.sum(pct[3 * b2:3 * b2 + 3] * pct[3 * b2:3 * b2 + 3],
                     axis=0, keepdims=True)
        p2r = jnp.where(bq == b2, pb, p2r)
    return q2 + p2r - 2.0 * cross, bq


def _argmin(dist, iota):
    m = jnp.min(dist, axis=1, keepdims=True)
    cand = jnp.where(dist <= m, iota, jnp.int32(0x7FFFFFFF))
    am = jnp.min(cand, axis=1, keepdims=True)
    return m, am


def _conv_kernel(s_ref, pct_ref, qp_ref, *wrefs,
                 n_layers, k_nn, d_feat, fold, nb, q_blk):
    wrefs, out_ref = wrefs[:-1], wrefs[-1]
    layers = [(wrefs[3 * i][...], wrefs[3 * i + 1][...], wrefs[3 * i + 2][...])
              for i in range(n_layers)]
    s = s_ref[0]
    pct = pct_ref[0]
    qp = qp_ref[0]
    n = pct.shape[1]
    qt = qp.shape[0]
    d = d_feat
    d_out = layers[-1][0].shape[1]

    dist, bq = _dist_stacked(qp, pct, nb, q_blk, pl.program_id(1) * qt)
    iota = jax.lax.broadcasted_iota(jnp.int32, (qt, n), 1)
    if fold:
        iota_c = jax.lax.broadcasted_iota(jnp.int32, (qt, nb * n // 8), 1)
    else:
        iota_g = jax.lax.broadcasted_iota(jnp.int32, (qt, nb * n), 1)
    s_parts = _bsplit(s)
    w1 = layers[0][0]

    def body(_, carry):
        dist, acc = carry
        _, am = _argmin(dist, iota)
        am_g = am if nb == 1 else am + bq * n
        if fold:
            rowhot = (iota_c == jax.lax.shift_right_logical(am_g, 3))
            t = _dot2(rowhot.astype(jnp.float32), s_parts)
            sub = jnp.bitwise_and(am_g, 7)
            g = t[:, 0:d]
            for j in range(1, 8):
                g = jnp.where(sub == j, t[:, j * d:(j + 1) * d], g)
        else:
            g = _dot2((iota_g == am_g).astype(jnp.float32), s_parts)
        g_cat = jnp.concatenate([g[:, :3] - qp, g[:, 3:]], axis=1)
        h = jnp.maximum(_dotd(g_cat, w1) * layers[0][1] + layers[0][2], 0.0)
        for w, ga, be in layers[1:]:
            h = jnp.maximum(_dotd(h, w) * ga + be, 0.0)
        acc = jnp.maximum(acc, h)
        dist = jnp.where(iota == am, _BIG, dist)
        return dist, acc

    _, acc = jax.lax.fori_loop(
        0, k_nn, body, (dist, jnp.zeros((qt, d_out), jnp.float32)),
        unroll=4)
    out_ref[0] = acc


def _conv_stage(s, pct, qp, layers, k_nn, q_tile, stack=False):
    b, n, d = s.shape
    q = qp.shape[1]
    d_out = layers[-1][0].shape[1]
    nb = b if stack else 1
    if stack:
        s = s.reshape(1, b * n, d)
        pct = pct.reshape(1, 3 * b, n)
        qp = qp.reshape(1, b * q, 3)
    bq_tot = qp.shape[1]
    q_tile = min(q_tile, bq_tot)
    fold = nb * n >= 1024
    if fold:
        s = s.reshape(s.shape[0], nb * n // 8, 8 * d)
    wargs = []
    for (w, ga, be) in layers:
        wargs += [w, ga.reshape(1, -1), be.reshape(1, -1)]
    grid = (s.shape[0], bq_tot // q_tile)
    full = lambda arr: pl.BlockSpec(arr.shape, lambda bi, qi: (0,) * arr.ndim)
    out = pl.pallas_call(
        functools.partial(_conv_kernel, n_layers=len(layers), k_nn=k_nn,
                          d_feat=d, fold=fold, nb=nb, q_blk=q),
        grid=grid,
        in_specs=[
            pl.BlockSpec((1,) + s.shape[1:], lambda bi, qi: (bi, 0, 0)),
            pl.BlockSpec((1,) + pct.shape[1:], lambda bi, qi: (bi, 0, 0)),
            pl.BlockSpec((1, q_tile, 3), lambda bi, qi: (bi, qi, 0)),
        ] + [full(a) for a in wargs],
        out_specs=pl.BlockSpec((1, q_tile, d_out), lambda bi, qi: (bi, qi, 0)),
        out_shape=jax.ShapeDtypeStruct((s.shape[0], bq_tot, d_out), jnp.float32),
    )(s, pct, qp, *wargs)
    return out.reshape(b, q, d_out)


def _knn_kernel(pct_ref, qp_ref, idx_ref, *, k_nn):
    pct = pct_ref[0]
    qp = qp_ref[0]
    n = pct.shape[1]
    qt = qp.shape[0]
    dist, _ = _dist_stacked(qp, pct, 1, 0, 0)
    iota = jax.lax.broadcasted_iota(jnp.int32, (qt, n), 1)
    kio = jax.lax.broadcasted_iota(jnp.int32, (qt, k_nn), 1)
    off = pl.program_id(0) * n

    def body(k, carry):
        dist, idxs = carry
        _, am = _argmin(dist, iota)
        idxs = jnp.where(kio == k, am + off, idxs)
        dist = jnp.where(iota == am, _BIG, dist)
        return dist, idxs

    _, idxs = jax.lax.fori_loop(
        0, k_nn, body, (dist, jnp.zeros((qt, k_nn), jnp.int32)), unroll=4)
    idx_ref[0] = idxs


def _knn_stage(pct, qp, k_nn, q_tile):
    b = pct.shape[0]
    n = pct.shape[2]
    q = qp.shape[1]
    q_tile = min(q_tile, q)
    return pl.pallas_call(
        functools.partial(_knn_kernel, k_nn=k_nn),
        grid=(b, q // q_tile),
        in_specs=[pl.BlockSpec((1, 3, n), lambda bi, qi: (bi, 0, 0)),
                  pl.BlockSpec((1, q_tile, 3), lambda bi, qi: (bi, qi, 0))],
        out_specs=pl.BlockSpec((1, q_tile, k_nn), lambda bi, qi: (bi, qi, 0)),
        out_shape=jax.ShapeDtypeStruct((b, q, k_nn), jnp.int32),
    )(pct, qp)


def _sc_gather(table, idx_flat, window=128):
    m = idx_flat.shape[0]
    d = table.shape[1]
    mesh = plsc.VectorSubcoreMesh(core_axis_name='core',
                                  subcore_axis_name='subcore',
                                  num_cores=2, num_subcores=16)
    idx2 = idx_flat.reshape(1, m)

    @functools.partial(
        pl.kernel,
        out_type=jax.ShapeDtypeStruct((m, d), table.dtype),
        mesh=mesh)
    def gk(x_hbm, i_hbm, o_hbm):
        def body(i_vmem, o_vmem):
            pltpu.sync_copy(x_hbm.at[i_vmem.at[0]], o_vmem)

        pltpu.emit_pipeline(
            body,
            grid=(m // window,),
            in_specs=[pl.BlockSpec((1, window), index_map=lambda i: (0, i))],
            out_specs=[pl.BlockSpec((window, d), index_map=lambda i: (i, 0))],
            core_axis_name='subcore',
            dimension_semantics=(pltpu.PARALLEL,),
        )(i_hbm, o_hbm)

    return gk(table, idx2)


def _mlpmax_kernel(g_ref, qp_ref, *wrefs, n_layers, k_nn, d_feat):
    wrefs, out_ref = wrefs[:-1], wrefs[-1]
    layers = [(wrefs[3 * i][...], wrefs[3 * i + 1][...], wrefs[3 * i + 2][...])
              for i in range(n_layers)]
    g = g_ref[0]
    qp = qp_ref[0]
    qt = qp.shape[0]
    g3 = g.reshape(qt, k_nn, g.shape[1])
    gc = jnp.concatenate([g3[:, :, :3] - qp[:, None, :],
                          g3[:, :, 3:d_feat]], axis=2)
    h = gc.reshape(qt * k_nn, d_feat)
    for w, ga, be in layers:
        h = jnp.maximum(_dotd(h, w) * ga + be, 0.0)
    d_out = layers[-1][0].shape[1]
    out_ref[0] = jnp.max(h.reshape(qt, k_nn, d_out), axis=1)


def _conv_stage_sc(s, pct, qp, layers, k_nn, q_tile, mlp_tile):
    b, n, d = s.shape
    q = qp.shape[1]
    d_out = layers[-1][0].shape[1]
    idx = _knn_stage(pct, qp, k_nn, q_tile)
    d_pad = 128
    s_pad = jnp.concatenate(
        [s, jnp.zeros((b, n, d_pad - d), jnp.float32)], axis=2)
    g = _sc_gather(s_pad.reshape(b * n, d_pad), idx.reshape(b * q * k_nn))
    g = g.reshape(b, q * k_nn, d_pad)
    wargs = []
    for (w, ga, be) in layers:
        wargs += [w, ga.reshape(1, -1), be.reshape(1, -1)]
    full = lambda arr: pl.BlockSpec(arr.shape, lambda bi, qi: (0,) * arr.ndim)
    return pl.pallas_call(
        functools.partial(_mlpmax_kernel, n_layers=len(layers), k_nn=k_nn,
                          d_feat=d),
        grid=(b, q // mlp_tile),
        in_specs=[
            pl.BlockSpec((1, mlp_tile * k_nn, d_pad),
                         lambda bi, qi: (bi, qi, 0)),
            pl.BlockSpec((1, mlp_tile, 3), lambda bi, qi: (bi, qi, 0)),
        ] + [full(a) for a in wargs],
        out_specs=pl.BlockSpec((1, mlp_tile, d_out), lambda bi, qi: (bi, qi, 0)),
        out_shape=jax.ShapeDtypeStruct((b, q, d_out), jnp.float32),
    )(g, qp, *wargs)


def _knn3_kernel(pct_ref, qp_ref, idx_ref, m_ref, *, k_nn):
    pct = pct_ref[0]
    qp = qp_ref[0]
    n = pct.shape[1]
    qt = qp.shape[0]
    dist, _ = _dist_stacked(qp, pct, 1, 0, 0)
    iota = jax.lax.broadcasted_iota(jnp.int32, (qt, n), 1)
    kio = jax.lax.broadcasted_iota(jnp.int32, (qt, k_nn), 1)
    off = pl.program_id(0) * n
    idxs = jnp.zeros((qt, k_nn), jnp.int32)
    ms = jnp.zeros((qt, k_nn), jnp.float32)
    for k in range(k_nn):
        m, am = _argmin(dist, iota)
        idxs = jnp.where(kio == k, am + off, idxs)
        ms = jnp.where(kio == k, m, ms)
        dist = jnp.where(iota == am, _BIG, dist)
    idx_ref[0] = idxs
    m_ref[0] = ms


def _knn3_stage(pct, qp, k_nn, q_tile):
    b = pct.shape[0]
    n = pct.shape[2]
    q = qp.shape[1]
    q_tile = min(q_tile, q)
    return pl.pallas_call(
        functools.partial(_knn3_kernel, k_nn=k_nn),
        grid=(b, q // q_tile),
        in_specs=[pl.BlockSpec((1, 3, n), lambda bi, qi: (bi, 0, 0)),
                  pl.BlockSpec((1, q_tile, 3), lambda bi, qi: (bi, qi, 0))],
        out_specs=[
            pl.BlockSpec((1, q_tile, k_nn), lambda bi, qi: (bi, qi, 0)),
            pl.BlockSpec((1, q_tile, k_nn), lambda bi, qi: (bi, qi, 0))],
        out_shape=[jax.ShapeDtypeStruct((b, q, k_nn), jnp.int32),
                   jax.ShapeDtypeStruct((b, q, k_nn), jnp.float32)],
    )(pct, qp)


def _dinterp_kernel(f_ref, m_ref, *wrefs, n_layers, k_nn, has_ff):
    wrefs, out_ref = wrefs[:-1], wrefs[-1]
    off = 0
    if has_ff:
        ff = wrefs[0][0]
        off = 1
    layers = [(wrefs[off + 3 * i][...], wrefs[off + 3 * i + 1][...],
               wrefs[off + 3 * i + 2][...]) for i in range(n_layers)]
    f = f_ref[0]
    ms = m_ref[0]
    cc = f.shape[1]
    qt = ms.shape[0]
    f3 = f.reshape(qt, k_nn, cc)
    w = 1.0 / jnp.maximum(ms, 1e-10)
    num = jnp.sum(f3 * w[:, :, None], axis=1)
    den = jnp.sum(w, axis=1, keepdims=True)
    interp = num / den
    w1, g1, b1 = layers[0]
    pre = _dotd(interp, w1[:cc, :])
    if has_ff:
        pre = pre + _dotd(ff, w1[cc:, :])
    h = jnp.maximum(pre * g1 + b1, 0.0)
    for w_, ga, be in layers[1:]:
        h = jnp.maximum(_dotd(h, w_) * ga + be, 0.0)
    out_ref[0] = h


def _deconv_stage_sc(fc, pct, qp, ff, layers, q_tile, mlp_tile):
    b, nc, cc = fc.shape
    q = qp.shape[1]
    d_out = layers[-1][0].shape[1]
    k_nn = 3
    idx, ms = _knn3_stage(pct, qp, k_nn, q_tile)
    g = _sc_gather(fc.reshape(b * nc, cc), idx.reshape(b * q * k_nn))
    g = g.reshape(b, q * k_nn, cc)
    wargs = []
    for (w, ga, be) in layers:
        wargs += [w, ga.reshape(1, -1), be.reshape(1, -1)]
    has_ff = ff is not None
    full = lambda arr: pl.BlockSpec(arr.shape, lambda bi, qi: (0,) * arr.ndim)
    in_specs = [
        pl.BlockSpec((1, mlp_tile * k_nn, cc), lambda bi, qi: (bi, qi, 0)),
        pl.BlockSpec((1, mlp_tile, k_nn), lambda bi, qi: (bi, qi, 0)),
    ]
    args = [g, ms]
    if has_ff:
        in_specs.append(pl.BlockSpec((1, mlp_tile, ff.shape[2]),
                                     lambda bi, qi: (bi, qi, 0)))
        args.append(ff)
    in_specs += [full(a) for a in wargs]
    args += wargs
    return pl.pallas_call(
        functools.partial(_dinterp_kernel, n_layers=len(layers), k_nn=k_nn,
                          has_ff=has_ff),
        grid=(b, q // mlp_tile),
        in_specs=in_specs,
        out_specs=pl.BlockSpec((1, mlp_tile, d_out), lambda bi, qi: (bi, qi, 0)),
        out_shape=jax.ShapeDtypeStruct((b, q, d_out), jnp.float32),
    )(*args)


def _deconv_kernel(fc_ref, pct_ref, qp_ref, *wrefs,
                   n_layers, has_ff, nb, q_blk):
    wrefs, out_ref = wrefs[:-1], wrefs[-1]
    off = 0
    if has_ff:
        ff = wrefs[0][0]
        off = 1
    layers = [(wrefs[off + 3 * i][...], wrefs[off + 3 * i + 1][...],
               wrefs[off + 3 * i + 2][...]) for i in range(n_layers)]
    fc = fc_ref[0]
    pct = pct_ref[0]
    qp = qp_ref[0]
    nc = pct.shape[1]
    cc = fc.shape[1]
    qt = qp.shape[0]

    dist, bq = _dist_stacked(qp, pct, nb, q_blk, pl.program_id(1) * qt)
    iota = jax.lax.broadcasted_iota(jnp.int32, (qt, nc), 1)
    iota_g = jax.lax.broadcasted_iota(jnp.int32, (qt, nb * nc), 1)
    fc_parts = _bsplit(fc)
    num = jnp.zeros((qt, cc), jnp.float32)
    den = jnp.zeros((qt, 1), jnp.float32)
    for _ in range(3):
        m, am = _argmin(dist, iota)
        am_g = am if nb == 1 else am + bq * nc
        f = _dot2((iota_g == am_g).astype(jnp.float32), fc_parts)
        w = 1.0 / jnp.maximum(m, 1e-10)
        num = num + w * f
        den = den + w
        dist = jnp.where(iota == am, _BIG, dist)
    interp = num / den

    w1, g1, b1 = layers[0]
    pre = _dotd(interp, w1[:cc, :])
    if has_ff:
        pre = pre + _dotd(ff, w1[cc:, :])
    h = jnp.maximum(pre * g1 + b1, 0.0)
    for w, ga, be in layers[1:]:
        h = jnp.maximum(_dotd(h, w) * ga + be, 0.0)
    out_ref[0] = h


def _deconv_stage(fc, pct, qp, ff, layers, q_tile, stack=False):
    b, nc, cc = fc.shape
    q = qp.shape[1]
    d_out = layers[-1][0].shape[1]
    nb = b if stack else 1
    if stack:
        fc = fc.reshape(1, b * nc, cc)
        pct = pct.reshape(1, 3 * b, nc)
        qp = qp.reshape(1, b * q, 3)
        if ff is not None:
            ff = ff.reshape(1, b * q, ff.shape[2])
    bq_tot = qp.shape[1]
    q_tile = min(q_tile, bq_tot)
    wargs = []
    for (w, ga, be) in layers:
        wargs += [w, ga.reshape(1, -1), be.reshape(1, -1)]
    has_ff = ff is not None
    grid = (fc.shape[0] if not stack else 1, bq_tot // q_tile)
    full = lambda arr: pl.BlockSpec(arr.shape, lambda bi, qi: (0,) * arr.ndim)
    in_specs = [
        pl.BlockSpec((1,) + fc.shape[1:], lambda bi, qi: (bi, 0, 0)),
        pl.BlockSpec((1,) + pct.shape[1:], lambda bi, qi: (bi, 0, 0)),
        pl.BlockSpec((1, q_tile, 3), lambda bi, qi: (bi, qi, 0)),
    ]
    args = [fc, pct, qp]
    if has_ff:
        in_specs.append(pl.BlockSpec((1, q_tile, ff.shape[2]),
                                     lambda bi, qi: (bi, qi, 0)))
        args.append(ff)
    in_specs += [full(a) for a in wargs]
    args += wargs
    out = pl.pallas_call(
        functools.partial(_deconv_kernel, n_layers=len(layers), has_ff=has_ff,
                          nb=nb, q_blk=q),
        grid=grid,
        in_specs=in_specs,
        out_specs=pl.BlockSpec((1, q_tile, d_out), lambda bi, qi: (bi, qi, 0)),
        out_shape=jax.ShapeDtypeStruct((fc.shape[0], bq_tot, d_out),
                                       jnp.float32),
    )(*args)
    return out.reshape(b, q, d_out)


def _head_kernel(f_ref, *wrefs, n_layers):
    wrefs, out_ref = wrefs[:-1], wrefs[-1]
    layers = [(wrefs[3 * i][...], wrefs[3 * i + 1][...], wrefs[3 * i + 2][...])
              for i in range(n_layers)]
    fcw = wrefs[3 * n_layers][...]
    fcb = wrefs[3 * n_layers + 1][...]
    h = f_ref[0]
    for w, ga, be in layers:
        h = jnp.maximum(_dotd(h, w) * ga + be, 0.0)
    out_ref[0] = _dotd(h, fcw) + fcb


def _head_stage(f, layers, fcw, fcb, q_tile):
    b, q, c = f.shape
    q_tile = min(q_tile, q)
    d_out = fcw.shape[1]
    wargs = []
    for (w, ga, be) in layers:
        wargs += [w, ga.reshape(1, -1), be.reshape(1, -1)]
    wargs += [fcw, fcb.reshape(1, -1)]
    full = lambda arr: pl.BlockSpec(arr.shape, lambda bi, qi: (0,) * arr.ndim)
    return pl.pallas_call(
        functools.partial(_head_kernel, n_layers=len(layers)),
        grid=(b, q // q_tile),
        in_specs=[pl.BlockSpec((1, q_tile, c), lambda bi, qi: (bi, qi, 0))]
                 + [full(a) for a in wargs],
        out_specs=pl.BlockSpec((1, q_tile, d_out), lambda bi, qi: (bi, qi, 0)),
        out_shape=jax.ShapeDtypeStruct((b, q, d_out), jnp.float32),
    )(f, *wargs)


def kernel(x, params):
    xt = jnp.transpose(x, (0, 2, 1))
    p0 = xt[..., :3]
    pct0 = x[:, :3]

    def ch(p):
        return jnp.transpose(p, (0, 2, 1))

    q1 = p0[:, ::4]
    f1 = _conv_stage_sc(xt, pct0, q1, params['conv1'], 32, 512, 256)
    s1 = jnp.concatenate([q1, f1], axis=-1)
    q2 = q1[:, ::4]
    f2 = _conv_stage_sc(s1, ch(q1), q2, params['conv2'], 32, 256, 256)
    s2 = jnp.concatenate([q2, f2], axis=-1)
    q3 = q2[:, ::4]
    f3 = _conv_stage(s2, ch(q2), q3, params['conv3'], 32, 256,
                     stack=True)
    s3 = jnp.concatenate([q3, f3], axis=-1)
    q4 = q3[:, ::4]
    f4 = _conv_stage(s3, ch(q3), q4, params['conv4'], 32, 64,
                     stack=True)

    g3 = _deconv_stage(f4, ch(q4), q3, f3, params['dconv1'], 256,
                       stack=True)
    g2 = _deconv_stage(g3, ch(q3), q2, f2, params['dconv2'], 1024,
                       stack=True)
    g1 = _deconv_stage(g2, ch(q2), q1, f1, params['dconv3'], 4096,
                       stack=True)
    g0 = _deconv_stage(g1, ch(q1), p0, None, params['dconv4'], 1024)

    out = _head_stage(g0, params['mlp'], params['fc_w'], params['fc_b'], 1024)
    return jnp.transpose(out, (0, 2, 1))
